# baseline (speedup 1.0000x reference)
import functools

import jax
import jax.numpy as jnp
import numpy as np
from jax import lax
from jax.experimental import pallas as pl
from jax.experimental.pallas import tpu as pltpu
from jax.experimental.pallas import tpu_sc as plsc

B = 16384
V = 27
JPAD = 32

_U32 = jnp.uint32
_K1 = np.uint32(0)
_K2 = np.uint32(42)
_K3 = np.uint32(0 ^ 42 ^ 0x1BD11BDA)
_TINY = np.float32(np.finfo(np.float32).tiny)

NW = 32
BPW = B // NW
NCHUNK = BPW // 16


def _rotl(x, r):
    return (x << _U32(r)) | (x >> _U32(32 - r))


def _threefry_bits(n):
    rotations = ((13, 15, 26, 6), (17, 29, 16, 24))
    ks = (_K1, _K2, _K3)
    x0 = jnp.zeros_like(n) + ks[0]
    x1 = n + ks[1]
    for i in range(5):
        for r in rotations[i % 2]:
            x0 = x0 + x1
            x1 = _rotl(x1, r)
            x1 = x0 ^ x1
        x0 = x0 + ks[(i + 1) % 3]
        x1 = x1 + ks[(i + 2) % 3] + _U32(i + 1)
    return x0 ^ x1


def _gumbel_from_bits(bits):
    fb = (bits >> _U32(9)) | _U32(0x3F800000)
    f = jax.lax.bitcast_convert_type(fb, jnp.float32) - jnp.float32(1.0)
    u = f * (jnp.float32(1.0) - _TINY) + _TINY
    u = jnp.maximum(_TINY, u)
    return -jnp.log(-jnp.log(u))


_sc_mesh = plsc.VectorSubcoreMesh(core_axis_name="c", subcore_axis_name="s")


@functools.partial(
    pl.kernel,
    out_type=jax.ShapeDtypeStruct((JPAD, B), jnp.float32),
    mesh=_sc_mesh,
    compiler_params=pltpu.CompilerParams(needs_layout_passes=False),
    scratch_types=[
        pltpu.VMEM((JPAD, JPAD), jnp.float32),
        pltpu.VMEM((BPW,), jnp.int32),
        pltpu.VMEM((JPAD, BPW), jnp.float32),
        pltpu.SemaphoreType.DMA,
    ],
)
def _sc_gather(tab_hbm, x_hbm, out_hbm, tab_v, xv_v, buf_v, sem):
    wid = lax.axis_index("s") * 2 + lax.axis_index("c")
    base = pl.multiple_of(wid * BPW, BPW)
    ctab = pltpu.async_copy(tab_hbm, tab_v, sem)
    cx = pltpu.async_copy(x_hbm.at[pl.ds(base, BPW)], xv_v, sem)
    ctab.wait()
    cx.wait()

    def chunk(c, carry):
        off = pl.multiple_of(c * 16, 16)
        xi = xv_v[pl.ds(off, 16)]
        for j in range(V):
            jv = jnp.full((16,), j, jnp.int32)
            vals = plsc.load_gather(tab_v, [jv, xi])
            buf_v[j, pl.ds(off, 16)] = vals
        return carry

    lax.fori_loop(0, NCHUNK, chunk, 0)
    pltpu.sync_copy(buf_v, out_hbm.at[:, pl.ds(base, BPW)])


def _tc_body(p_ref, out_ref):
    j = jax.lax.broadcasted_iota(jnp.int32, (JPAD, B), 0)
    i = jax.lax.broadcasted_iota(jnp.int32, (JPAD, B), 1)
    n = (i * V + j).astype(_U32)
    g = _gumbel_from_bits(_threefry_bits(n))
    scores = g + jnp.log(p_ref[...])
    scores = jnp.where(j < V, scores, -jnp.inf)
    val, idx = scores, j
    for size in (16, 8, 4, 2, 1):
        av, bv = val[:size], val[size:2 * size]
        ai, bi = idx[:size], idx[size:2 * size]
        takeb = (bv > av) | ((bv == av) & (bi < ai))
        val = jnp.where(takeb, bv, av)
        idx = jnp.where(takeb, bi, ai)
    out_ref[...] = idx


@jax.jit
def kernel(x, logits):
    lt = jnp.ones((JPAD, JPAD), jnp.float32).at[:V, :V].set(logits.T)
    probT = _sc_gather(lt, x.astype(jnp.int32))
    out = pl.pallas_call(
        _tc_body,
        out_shape=jax.ShapeDtypeStruct((1, B), jnp.int32),
    )(probT)
    return out.reshape(B, 1)


# speedup vs baseline: 2.8472x; 2.8472x over previous
import functools

import jax
import jax.numpy as jnp
import numpy as np
from jax.experimental import pallas as pl

B = 16384
V = 27
JPAD = 32

_U32 = jnp.uint32
_K1 = np.uint32(0)
_K2 = np.uint32(42)
_K3 = np.uint32(0 ^ 42 ^ 0x1BD11BDA)
_TINY = np.float32(np.finfo(np.float32).tiny)


def _rotl(x, r):
    return (x << _U32(r)) | (x >> _U32(32 - r))


def _threefry_bits(n):
    rotations = ((13, 15, 26, 6), (17, 29, 16, 24))
    keyadds = ((_K2, _K3 + np.uint32(1)), (_K3, np.uint32(2)),
               (None, _K2 + np.uint32(3)), (_K2, _K3 + np.uint32(4)),
               (_K3, np.uint32(5)))
    x1 = n + _K2
    x0 = x1
    x1 = x0 ^ _rotl(x1, 13)
    for r in (15, 26, 6):
        x0 = x0 + x1
        x1 = x0 ^ _rotl(x1, r)
    x0 = x0 + keyadds[0][0]
    x1 = x1 + keyadds[0][1]
    for i in range(1, 5):
        for r in rotations[i % 2]:
            x0 = x0 + x1
            x1 = x0 ^ _rotl(x1, r)
        c0, c1 = keyadds[i]
        if c0 is not None:
            x0 = x0 + c0
        x1 = x1 + c1
    return x0 ^ x1


def _gumbel_from_bits(bits):
    fb = (bits >> _U32(9)) | _U32(0x3F800000)
    f = jax.lax.bitcast_convert_type(fb, jnp.float32) - jnp.float32(1.0)
    u = f * (jnp.float32(1.0) - _TINY) + _TINY
    u = jnp.maximum(_TINY, u)
    return -jnp.log(-jnp.log(u))


def _body(x_ref, lt_ref, out_ref):
    j = jax.lax.broadcasted_iota(jnp.int32, (JPAD, B), 0)
    i = jax.lax.broadcasted_iota(jnp.int32, (JPAD, B), 1)
    n = (i * V + j).astype(_U32)
    g = _gumbel_from_bits(_threefry_bits(n))

    jt = jax.lax.broadcasted_iota(jnp.int32, (JPAD, JPAD), 0)
    tab = jnp.where(jt < V, jnp.log(lt_ref[...]), jnp.float32(-1e30))
    onehot = (j == x_ref[...]).astype(jnp.float32)
    logp = jax.lax.dot_general(
        tab, onehot, (((1,), (0,)), ((), ())),
        preferred_element_type=jnp.float32,
        precision=jax.lax.Precision.HIGHEST)

    scores = g + logp

    val, idx = scores, j
    for size in (16, 8, 4, 2, 1):
        av, bv = val[:size], val[size:2 * size]
        ai, bi = idx[:size], idx[size:2 * size]
        takeb = (bv > av) | ((bv == av) & (bi < ai))
        val = jnp.where(takeb, bv, av)
        idx = jnp.where(takeb, bi, ai)
    out_ref[...] = idx


@functools.partial(jax.jit, static_argnames=())
def kernel(x, logits):
    xr = x.reshape(1, B).astype(jnp.int32)
    lt = jnp.ones((JPAD, JPAD), jnp.float32).at[:V, :V].set(logits.T)
    out = pl.pallas_call(
        _body,
        out_shape=jax.ShapeDtypeStruct((1, B), jnp.int32),
    )(xr, lt)
    return out.reshape(B, 1)


# speedup vs baseline: 3.1892x; 1.1201x over previous
import functools

import jax
import jax.numpy as jnp
import numpy as np
from jax.experimental import pallas as pl

B = 16384
V = 27
JPAD = 32

_U32 = jnp.uint32
_K1 = np.uint32(0)
_K2 = np.uint32(42)
_K3 = np.uint32(0 ^ 42 ^ 0x1BD11BDA)
_TINY = np.float32(np.finfo(np.float32).tiny)


def _rotl(x, r):
    return (x << _U32(r)) | (x >> _U32(32 - r))


def _threefry_bits(n):
    rotations = ((13, 15, 26, 6), (17, 29, 16, 24))
    keyadds = ((_K2, _K3 + np.uint32(1)), (_K3, np.uint32(2)),
               (None, _K2 + np.uint32(3)), (_K2, _K3 + np.uint32(4)),
               (_K3, np.uint32(5)))
    x1 = n + _K2
    x0 = x1
    x1 = x0 ^ _rotl(x1, 13)
    for r in (15, 26, 6):
        x0 = x0 + x1
        x1 = x0 ^ _rotl(x1, r)
    x0 = x0 + keyadds[0][0]
    x1 = x1 + keyadds[0][1]
    for i in range(1, 5):
        for r in rotations[i % 2]:
            x0 = x0 + x1
            x1 = x0 ^ _rotl(x1, r)
        c0, c1 = keyadds[i]
        if c0 is not None:
            x0 = x0 + c0
        x1 = x1 + c1
    return x0 ^ x1


def _gumbel_from_bits(bits):
    fb = (bits >> _U32(9)) | _U32(0x3F800000)
    f = jax.lax.bitcast_convert_type(fb, jnp.float32) - jnp.float32(1.0)
    u = f * (jnp.float32(1.0) - _TINY) + _TINY
    u = jnp.maximum(_TINY, u)
    return -jnp.log(-jnp.log(u))


def _body(x_ref, lt_ref, out_ref):
    j = jax.lax.broadcasted_iota(jnp.int32, (JPAD, B), 0)
    i = jax.lax.broadcasted_iota(jnp.int32, (JPAD, B), 1)
    n = (i * V + j).astype(_U32)
    g = _gumbel_from_bits(_threefry_bits(n))

    tab = jnp.log(lt_ref[...])
    tab = jnp.concatenate([tab, jnp.full((V, JPAD - V), -1e30, jnp.float32)],
                          axis=1)
    onehot = (j == x_ref[...]).astype(jnp.float32)[:V]
    logp = jax.lax.dot_general(
        tab, onehot, (((0,), (0,)), ((), ())),
        preferred_element_type=jnp.float32,
        precision=jax.lax.Precision.HIGHEST)

    scores = g + logp

    val, idx = scores, j
    for size in (16, 8, 4, 2, 1):
        av, bv = val[:size], val[size:2 * size]
        ai, bi = idx[:size], idx[size:2 * size]
        takeb = (bv > av) | ((bv == av) & (bi < ai))
        val = jnp.where(takeb, bv, av)
        idx = jnp.where(takeb, bi, ai)
    out_ref[...] = idx


@functools.partial(jax.jit, static_argnames=())
def kernel(x, logits):
    xr = x.reshape(1, B).astype(jnp.int32)
    out = pl.pallas_call(
        _body,
        out_shape=jax.ShapeDtypeStruct((1, B), jnp.int32),
    )(xr, logits)
    return out.reshape(B, 1)
